# baseline (speedup 1.0000x reference)
import functools

import jax
import jax.numpy as jnp
from jax import lax
from jax.experimental import pallas as pl
from jax.experimental.pallas import tpu as pltpu
from jax.experimental.pallas import tpu_sc as plsc

N, E, DIN, DOUT = 10000, 160000, 256, 256
HALF = DOUT // 2
NC, NS, L = 2, 16, 16
PER_TEC = E // NS
CH = 80
STG = 2000
NSTG = PER_TEC // STG
NCH = STG // CH
RPT = 624
RPT_LAST = N - 15 * RPT


_BN = 2000


def _matmul_body(x_ref, w_ref, b_ref, h_ref):
    h_ref[...] = (
        jnp.dot(x_ref[...], w_ref[...], preferred_element_type=jnp.float32)
        + b_ref[...]
    ).astype(jnp.bfloat16)


def _project(x, W, b2):
    return pl.pallas_call(
        _matmul_body,
        grid=(NC, N // _BN),
        in_specs=[
            pl.BlockSpec((_BN, DIN), lambda h, i: (i, 0)),
            pl.BlockSpec((DIN, HALF), lambda h, i: (0, h)),
            pl.BlockSpec((1, HALF), lambda h, i: (0, h)),
        ],
        out_specs=pl.BlockSpec((_BN, HALF), lambda h, i: (h * (N // _BN) + i, 0)),
        out_shape=jax.ShapeDtypeStruct((NC * N, HALF), jnp.bfloat16),
    )(x, W, b2)


_sc_mesh = plsc.VectorSubcoreMesh(core_axis_name="c", subcore_axis_name="s")


NB = 4
NBI = 8
NTOT = PER_TEC // CH
NOCT = (NTOT - 5) // NBI


@functools.partial(
    pl.kernel,
    mesh=_sc_mesh,
    compiler_params=pltpu.CompilerParams(
        needs_layout_passes=False, use_tc_tiling_on_sc=False),
    out_type=jax.ShapeDtypeStruct((NC * N, HALF), jnp.float32),
    scratch_types=(
        [pltpu.VMEM((CH,), jnp.int32) for _ in range(NBI)]
        + [pltpu.VMEM((CH,), jnp.int32) for _ in range(NBI)]
        + [pltpu.VMEM((CH,), jnp.float32) for _ in range(NBI)]
        + [pltpu.VMEM((CH, HALF // 2), jnp.int32) for _ in range(NB)]
        + [pltpu.VMEM((CH, HALF), jnp.float32) for _ in range(2)]
        + [pltpu.VMEM_SHARED((N, HALF), jnp.float32)]
        + [pltpu.SemaphoreType.DMA for _ in range(2 * NB + NBI)]
        + [pltpu.SemaphoreType.DMA for _ in range(2)]
    ),
)
def _sc_aggregate(h2, col, row, ew, out, *sc):
    cb, rb, wb = sc[0:NBI], sc[NBI:2 * NBI], sc[2 * NBI:3 * NBI]
    gb = sc[3 * NBI:3 * NBI + NB]
    ob = sc[3 * NBI + NB:3 * NBI + NB + 2]
    acc = sc[3 * NBI + NB + 2]
    base_s = 3 * NBI + NB + 3
    gsem = sc[base_s:base_s + NB]
    isem = sc[base_s + NB:base_s + NB + NBI]
    ssem = sc[base_s + NB + NBI:base_s + NB + NBI + 2]

    c = lax.axis_index("c")
    s = lax.axis_index("s")
    zeros = jnp.zeros((L,), jnp.float32)
    off = c * N

    def idx_issue(i, b):
        base = pl.multiple_of(s * PER_TEC + i * CH, 8)
        pltpu.async_copy(col.at[pl.ds(base, CH)], cb[b], isem[b])
        pltpu.async_copy(row.at[pl.ds(base, CH)], rb[b], isem[b])
        pltpu.async_copy(ew.at[pl.ds(base, CH)], wb[b], isem[b])

    def idx_wait_bias(b):
        pltpu.make_async_copy(col.at[pl.ds(0, CH)], cb[b], isem[b]).wait()
        pltpu.make_async_copy(row.at[pl.ds(0, CH)], rb[b], isem[b]).wait()
        pltpu.make_async_copy(ew.at[pl.ds(0, CH)], wb[b], isem[b]).wait()
        for k in range(CH // L):
            o = pl.multiple_of(k * L, L)
            cb[b][pl.ds(o, L)] = cb[b][pl.ds(o, L)] + off

    def gather_issue(ib, b):
        pltpu.async_copy(h2.at[cb[ib]], gb[b], gsem[b])

    def gather_wait(ib, b):
        pltpu.make_async_copy(h2.at[cb[ib]], gb[b], gsem[b]).wait()

    def scatter_issue(ib, o):
        pltpu.async_copy(ob[o], acc.at[rb[ib]], ssem[o], add=True)

    def scatter_wait(ib, o):
        pltpu.make_async_copy(ob[o], acc.at[rb[ib]], ssem[o]).wait()

    def scale(ib, b, o):
        gbuf, obuf, wbuf = gb[b], ob[o], wb[ib]

        @pl.loop(0, CH // L)
        def _scale(m):
            wv16 = wbuf[pl.ds(pl.multiple_of(m * L, L), L)]
            for t in range(L):
                wv = jnp.full((L,), wv16[t])
                e = m * L + t
                for j in range(HALF // (2 * L)):
                    w32 = gbuf[e, pl.ds(j * L, L)]
                    v32 = plsc.bitcast(w32, jnp.bfloat16)
                    a, bb = plsc.unpack(v32, format=plsc.PackFormat.INTERLEAVED)
                    obuf[e, pl.ds(2 * j * L, L)] = a * wv
                    obuf[e, pl.ds((2 * j + 1) * L, L)] = bb * wv


    def step(i, r8, do_swait, swait_q, do_idx, do_prep):
        r4 = r8 % NB
        o2 = r8 % 2
        gather_wait(r8 % NBI, r4)
        sw_ib, sw_b = (r8 - 2) % NBI, (r8 - 2) % 2
        if do_swait:
            if swait_q is None:
                scatter_wait(sw_ib, sw_b)
            else:
                @pl.when(swait_q)
                def _sw():
                    scatter_wait(sw_ib, sw_b)
        scale(r8 % NBI, r4, o2)
        scatter_issue(r8 % NBI, o2)
        if do_idx:
            idx_issue(i + NB, (r8 + NB) % NBI)
        if do_prep:
            idx_wait_bias((r8 + 2) % NBI)
            gather_issue((r8 + 2) % NBI, (r8 + 2) % NB)

    for i in range(NB):
        idx_issue(i, i)
    for i in range(NB - 2):
        idx_wait_bias(i)
        gather_issue(i, i)

    @pl.loop(0, CH)
    def _zero(r):
        for j in range(HALF // L):
            ob[1][r, pl.ds(j * L, L)] = zeros

    r0 = pl.multiple_of(s * RPT, 8)
    for k in range(RPT // CH):
        pltpu.sync_copy(ob[1], acc.at[pl.ds(r0 + k * CH, CH)])
    rem = RPT - (RPT // CH) * CH
    pltpu.sync_copy(ob[1].at[pl.ds(0, rem)],
                    acc.at[pl.ds(r0 + (RPT // CH) * CH, rem)])

    @pl.when(s == NS - 1)
    def _zero_tail():
        pltpu.sync_copy(
            ob[1].at[pl.ds(0, RPT_LAST - RPT)],
            acc.at[pl.ds(r0 + RPT, RPT_LAST - RPT)],
        )

    plsc.subcore_barrier()

    @pl.loop(0, NOCT)
    def _oct(q):
        for r in range(NBI):
            i = q * NBI + r
            step(i, r, True, (q > 0) if r < 2 else None, True, True)

    t0 = NOCT * NBI
    step(t0 + 0, (t0 + 0) % NBI, True, None, True, True)
    step(t0 + 1, (t0 + 1) % NBI, True, None, False, True)
    step(t0 + 2, (t0 + 2) % NBI, True, None, False, True)
    step(t0 + 3, (t0 + 3) % NBI, True, None, False, False)
    step(t0 + 4, (t0 + 4) % NBI, True, None, False, False)
    scatter_wait((NTOT - 2) % NBI, (NTOT - 2) % 2)
    scatter_wait((NTOT - 1) % NBI, (NTOT - 1) % 2)

    plsc.subcore_barrier()

    c0 = pl.multiple_of(c * N + r0, 8)

    @pl.when(s < NS - 1)
    def _write_body():
        pltpu.sync_copy(acc.at[pl.ds(r0, RPT)], out.at[pl.ds(c0, RPT)])

    @pl.when(s == NS - 1)
    def _write_tail():
        pltpu.sync_copy(acc.at[pl.ds(r0, RPT_LAST)],
                        out.at[pl.ds(c0, RPT_LAST)])


def kernel(x, edge_index, edge_weight, W, b):
    Wp = W.reshape(DIN, DOUT // 32, 2, 16).transpose(0, 1, 3, 2).reshape(DIN, DOUT)
    bp = b.reshape(DOUT // 32, 2, 16).transpose(0, 2, 1).reshape(1, DOUT)
    h2 = _project(x, Wp, bp)
    h2i = jax.lax.bitcast_convert_type(
        h2.reshape(NC * N, HALF // 2, 2), jnp.int32)
    out2 = _sc_aggregate(h2i, edge_index[1], edge_index[0], edge_weight)
    return jnp.concatenate([out2[:N], out2[N:]], axis=1)


# speedup vs baseline: 2.5592x; 2.5592x over previous
import functools

import jax
import jax.numpy as jnp
from jax import lax
from jax.experimental import pallas as pl
from jax.experimental.pallas import tpu as pltpu
from jax.experimental.pallas import tpu_sc as plsc

N, E, DIN, DOUT = 10000, 160000, 256, 256
HALF = DOUT // 2
NC, NS, L = 2, 16, 16
PER_TEC = E // NS
CH = 80
STG = 2000
NSTG = PER_TEC // STG
NCH = STG // CH
RPT = 624
RPT_LAST = N - 15 * RPT


_BN = 2000


def _matmul_body(x_ref, w_ref, b_ref, h_ref):
    h_ref[...] = (
        jnp.dot(x_ref[...], w_ref[...], preferred_element_type=jnp.float32)
        + b_ref[...]
    )


def _project(x, W, b2):
    return pl.pallas_call(
        _matmul_body,
        grid=(NC, N // _BN),
        in_specs=[
            pl.BlockSpec((_BN, DIN), lambda h, i: (i, 0)),
            pl.BlockSpec((DIN, HALF), lambda h, i: (0, h)),
            pl.BlockSpec((1, HALF), lambda h, i: (0, h)),
        ],
        out_specs=pl.BlockSpec((_BN, HALF), lambda h, i: (h * (N // _BN) + i, 0)),
        out_shape=jax.ShapeDtypeStruct((NC * N, HALF), jnp.float32),
    )(x, W, b2)


_sc_mesh = plsc.VectorSubcoreMesh(core_axis_name="c", subcore_axis_name="s")


NB = 4
NBI = 8
NTOT = PER_TEC // CH
NOCT = (NTOT - 5) // NBI


@functools.partial(
    pl.kernel,
    mesh=_sc_mesh,
    out_type=jax.ShapeDtypeStruct((N, DOUT), jnp.float32),
    scratch_types=(
        [pltpu.VMEM((CH,), jnp.int32) for _ in range(NBI)]
        + [pltpu.VMEM((CH,), jnp.int32) for _ in range(NBI)]
        + [pltpu.VMEM((CH,), jnp.float32) for _ in range(NBI)]
        + [pltpu.VMEM((CH, HALF), jnp.float32) for _ in range(NB)]
        + [pltpu.VMEM_SHARED((N, HALF), jnp.float32)]
        + [pltpu.SemaphoreType.DMA for _ in range(2 * NB + NBI)]
    ),
)
def _sc_aggregate(h2, col, row, ew, out, *sc):
    cb, rb, wb = sc[0:NBI], sc[NBI:2 * NBI], sc[2 * NBI:3 * NBI]
    gb = sc[3 * NBI:3 * NBI + NB]
    acc = sc[3 * NBI + NB]
    base_s = 3 * NBI + NB + 1
    gsem = sc[base_s:base_s + NB]
    isem = sc[base_s + NB:base_s + NB + NBI]
    ssem = sc[base_s + NB + NBI:base_s + NB + NBI + NB]

    c = lax.axis_index("c")
    s = lax.axis_index("s")
    zeros = jnp.zeros((L,), jnp.float32)
    off = c * N

    def idx_issue(i, b):
        base = pl.multiple_of(s * PER_TEC + i * CH, 8)
        pltpu.async_copy(col.at[pl.ds(base, CH)], cb[b], isem[b])
        pltpu.async_copy(row.at[pl.ds(base, CH)], rb[b], isem[b])
        pltpu.async_copy(ew.at[pl.ds(base, CH)], wb[b], isem[b])

    def idx_wait_bias(b):
        pltpu.make_async_copy(col.at[pl.ds(0, CH)], cb[b], isem[b]).wait()
        pltpu.make_async_copy(row.at[pl.ds(0, CH)], rb[b], isem[b]).wait()
        pltpu.make_async_copy(ew.at[pl.ds(0, CH)], wb[b], isem[b]).wait()
        for k in range(CH // L):
            o = pl.multiple_of(k * L, L)
            cb[b][pl.ds(o, L)] = cb[b][pl.ds(o, L)] + off

    def gather_issue(ib, b):
        pltpu.async_copy(h2.at[cb[ib]], gb[b], gsem[b])

    def gather_wait(ib, b):
        pltpu.make_async_copy(h2.at[cb[ib]], gb[b], gsem[b]).wait()

    def scatter_issue(ib, b):
        pltpu.async_copy(gb[b], acc.at[rb[ib]], ssem[b], add=True)

    def scatter_wait(ib, b):
        pltpu.make_async_copy(gb[b], acc.at[rb[ib]], ssem[b]).wait()

    def scale(ib, b):
        gbuf, wbuf = gb[b], wb[ib]

        @pl.loop(0, CH // L)
        def _scale(m):
            wv16 = wbuf[pl.ds(pl.multiple_of(m * L, L), L)]
            for t in range(L):
                wv = jnp.full((L,), wv16[t])
                e = m * L + t
                for j in range(HALF // L):
                    gbuf[e, pl.ds(j * L, L)] = gbuf[e, pl.ds(j * L, L)] * wv


    def step(i, r8, do_swait, swait_q, do_idx, do_prep):
        r4 = r8 % NB
        gather_wait(r8 % NBI, r4)
        scale(r8 % NBI, r4)
        scatter_issue(r8 % NBI, r4)
        sw_ib, sw_b = (r8 - 2) % NBI, (r8 - 2) % NB
        if do_swait:
            if swait_q is None:
                scatter_wait(sw_ib, sw_b)
            else:
                @pl.when(swait_q)
                def _sw():
                    scatter_wait(sw_ib, sw_b)
        if do_idx:
            idx_issue(i + NB, (r8 + NB) % NBI)
        if do_prep:
            idx_wait_bias((r8 + 2) % NBI)
            gather_issue((r8 + 2) % NBI, (r8 + 2) % NB)

    for i in range(NB):
        idx_issue(i, i)
    for i in range(NB - 2):
        idx_wait_bias(i)
        gather_issue(i, i)

    @pl.loop(0, CH)
    def _zero(r):
        for j in range(HALF // L):
            gb[NB - 1][r, pl.ds(j * L, L)] = zeros

    r0 = pl.multiple_of(s * RPT, 8)
    for k in range(RPT // CH):
        pltpu.sync_copy(gb[NB - 1], acc.at[pl.ds(r0 + k * CH, CH)])
    rem = RPT - (RPT // CH) * CH
    pltpu.sync_copy(gb[NB - 1].at[pl.ds(0, rem)],
                    acc.at[pl.ds(r0 + (RPT // CH) * CH, rem)])

    @pl.when(s == NS - 1)
    def _zero_tail():
        pltpu.sync_copy(
            gb[NB - 1].at[pl.ds(0, RPT_LAST - RPT)],
            acc.at[pl.ds(r0 + RPT, RPT_LAST - RPT)],
        )

    plsc.subcore_barrier()

    @pl.loop(0, NOCT)
    def _oct(q):
        for r in range(NBI):
            i = q * NBI + r
            step(i, r, True, (q > 0) if r < 2 else None, True, True)

    t0 = NOCT * NBI
    step(t0 + 0, (t0 + 0) % NBI, True, None, True, True)
    step(t0 + 1, (t0 + 1) % NBI, True, None, False, True)
    step(t0 + 2, (t0 + 2) % NBI, True, None, False, True)
    step(t0 + 3, (t0 + 3) % NBI, True, None, False, False)
    step(t0 + 4, (t0 + 4) % NBI, True, None, False, False)
    scatter_wait((NTOT - 2) % NBI, (NTOT - 2) % NB)
    scatter_wait((NTOT - 1) % NBI, (NTOT - 1) % NB)

    plsc.subcore_barrier()

    c0 = pl.multiple_of(c * HALF, HALF)

    @pl.when(s < NS - 1)
    def _write_body():
        pltpu.sync_copy(
            acc.at[pl.ds(r0, RPT)],
            out.at[pl.ds(r0, RPT), pl.ds(c0, HALF)],
        )

    @pl.when(s == NS - 1)
    def _write_tail():
        pltpu.sync_copy(
            acc.at[pl.ds(r0, RPT_LAST)],
            out.at[pl.ds(r0, RPT_LAST), pl.ds(c0, HALF)],
        )


def kernel(x, edge_index, edge_weight, W, b):
    h2 = _project(x, W, b.reshape(1, DOUT))
    return _sc_aggregate(h2, edge_index[1], edge_index[0], edge_weight)
